# baseline (speedup 1.0000x reference)
import jax
import jax.numpy as jnp
from jax import lax
from jax.experimental import pallas as pl
from jax.experimental.pallas import tpu as pltpu
from jax.experimental.pallas import tpu_sc as plsc

N = 10000
E = 160000
IN_CH = 256
OUT_CH = 64
HEADS = 8
HC = HEADS * OUT_CH
NCHUNK = 8
CW = HC // NCHUNK

NC, NS, L = 2, 16, 16
NW = NC * NS
ETRUE = E + N
BATCH = 128
EB_PER_W = 42
EB_C0 = 52
EB_C1 = 2 * EB_PER_W - EB_C0
EBMAX = max(EB_C0, EB_C1)
NBAT = NS * (EB_C0 + EB_C1)
EPAD = NBAT * BATCH
NBAT_ALLOC = NBAT + EBMAX - min(EB_C0, EB_C1)
NPAD = 10240
RPT = NPAD // NS
EPW = EB_PER_W * BATCH

_ROWS = 1000
_GRID = N // _ROWS


def _proj_body(x_ref, w_ref, a_ref, *out_refs):
    hrefs = out_refs[:NCHUNK]
    acat_ref = out_refs[NCHUNK]
    macc_ref = out_refs[NCHUNK + 1]
    i = pl.program_id(0)
    h = jnp.dot(x_ref[...], w_ref[...], preferred_element_type=jnp.float32)
    for c in range(NCHUNK):
        hrefs[c][...] = h[:, c * CW:(c + 1) * CW]
    acat = jnp.dot(h, a_ref[...], preferred_element_type=jnp.float32)
    acat_ref[...] = acat
    bm = jnp.max(acat, axis=0, keepdims=True)

    @pl.when(i == 0)
    def _():
        macc_ref[...] = bm

    @pl.when(i != 0)
    def _():
        macc_ref[...] = jnp.maximum(macc_ref[...], bm)


def _project(x, W, A):
    return pl.pallas_call(
        _proj_body,
        grid=(_GRID,),
        in_specs=[
            pl.BlockSpec((_ROWS, IN_CH), lambda i: (i, 0)),
            pl.BlockSpec((IN_CH, HC), lambda i: (0, 0)),
            pl.BlockSpec((HC, 16), lambda i: (0, 0)),
        ],
        out_specs=[pl.BlockSpec((_ROWS, CW), lambda i: (i, 0))] * NCHUNK
        + [
            pl.BlockSpec((_ROWS, 16), lambda i: (i, 0)),
            pl.BlockSpec((1, 16), lambda i: (0, 0)),
        ],
        out_shape=[jax.ShapeDtypeStruct((N, CW), jnp.float32)] * NCHUNK
        + [
            jax.ShapeDtypeStruct((N, 16), jnp.float32),
            jax.ShapeDtypeStruct((1, 16), jnp.float32),
        ],
    )(x, W, A)


def _sc_body(acat, src2, dst2, macc, *rest):
    hrefs = rest[:NCHUNK]
    seg_out, msg_out, p_out = rest[NCHUNK:NCHUNK + 3]
    (sidx2, didx2, asrc0, asrc1, adst0, adst1, sbuf0, sbuf1, mvec,
     hbuf0, hbuf1, hbuf2, hbuf3, pbuf0, pbuf1, pbuf2, pbuf3,
     zb16, zb64, seg_acc, msg_acc,
     sg0, sg1, sg2, sg3, ss0, ss1, ss2, ss3) = rest[NCHUNK + 3:]
    c_idx = lax.axis_index("c")
    s_idx = lax.axis_index("s")

    eb_c = jnp.where(c_idx == 0, EB_C0, EB_C1)
    bbase = jnp.where(c_idx == 0, s_idx * EB_C0, NS * EB_C0 + s_idx * EB_C1)
    ebase = bbase * BATCH

    lanes = lax.broadcasted_iota(jnp.int32, (16,), 0)
    rot = lax.rem(lanes + 8, jnp.full((16,), 16, jnp.int32))

    def take(v, idx):
        return v.at[idx].get(mode="promise_in_bounds")

    pltpu.sync_copy(src2.at[pl.ds(bbase, EBMAX)], sidx2)
    pltpu.sync_copy(dst2.at[pl.ds(bbase, EBMAX)], didx2)

    pltpu.sync_copy(macc, mvec)
    mv = mvec[...]
    mb = mv + take(mv, rot)
    mb = jnp.where(mb >= 0, mb, 0.2 * mb)
    mfull = jnp.where(lanes < 8, mb, jnp.full((16,), 3.0e38, jnp.float32))

    def _zb_body(r, _):
        zb16[r] = jnp.zeros((16,), jnp.float32)
        for j in range(CW // L):
            zb64[r, pl.ds(j * L, L)] = jnp.zeros((16,), jnp.float32)
        return 0

    lax.fori_loop(0, RPT // 20, _zb_body, 0)

    def _zero_seg():
        for k in range(20):
            pltpu.sync_copy(
                zb16, seg_acc.at[pl.ds(s_idx * RPT + k * (RPT // 20), RPT // 20)])

    def _zero_msg():
        for k in range(20):
            pltpu.sync_copy(
                zb64, msg_acc.at[pl.ds(s_idx * RPT + k * (RPT // 20), RPT // 20)])

    _zero_seg()
    _zero_msg()
    plsc.subcore_barrier()

    asrc = (asrc0, asrc1)
    adst = (adst0, adst1)
    sbuf = (sbuf0, sbuf1)
    agsem = ((sg0, sg1), (sg2, sg3))
    assem = (ss0, ss1)
    apsem = (ss2, ss3)

    def _p1_gather(b, slot):
        pltpu.async_copy(acat.at[sidx2.at[b]], asrc[slot], agsem[slot][0])
        pltpu.async_copy(acat.at[didx2.at[b]], adst[slot], agsem[slot][1])

    _p1_gather(0, 0)

    def _p1_body(g, _):
        for slot in range(2):
            b = 2 * g + slot
            base = ebase + b * BATCH
            pltpu.make_async_copy(
                acat.at[sidx2.at[b]], asrc[slot], agsem[slot][0]).wait()
            pltpu.make_async_copy(
                acat.at[didx2.at[b]], adst[slot], agsem[slot][1]).wait()

            @pl.when(b + 1 < eb_c)
            def _():
                _p1_gather(b + 1, 1 - slot)

            @pl.when(g > 0)
            def _():
                pltpu.make_async_copy(
                    sbuf[slot], seg_acc.at[didx2.at[b]], assem[slot]).wait()
                pltpu.make_async_copy(
                    sbuf[slot], p_out.at[pl.ds(base, BATCH)],
                    apsem[slot]).wait()

            def q_of(e):
                al = asrc[slot][e] + take(adst[slot][e], rot)
                al = jnp.where(al >= 0, al, 0.2 * al)
                q = jnp.exp(al - mfull)
                valid = jnp.where(base + e < ETRUE, 1.0, 0.0)
                return q * valid

            def _quad(i, _):
                for u in range(4):
                    e = 4 * i + u
                    sbuf[slot][e] = q_of(e)
                return 0

            lax.fori_loop(0, BATCH // 4, _quad, 0)
            pltpu.async_copy(
                sbuf[slot], p_out.at[pl.ds(base, BATCH)], apsem[slot])
            pltpu.async_copy(
                sbuf[slot], seg_acc.at[didx2.at[b]], assem[slot], add=True)
        return 0

    lax.fori_loop(0, eb_c // 2, _p1_body, 0)
    for slot in range(2):
        b_last = eb_c - 2 + slot
        pltpu.make_async_copy(
            sbuf[slot], seg_acc.at[didx2.at[b_last]], assem[slot]).wait()
        pltpu.make_async_copy(
            sbuf[slot], p_out.at[pl.ds(ebase + b_last * BATCH, BATCH)],
            apsem[slot]).wait()
    plsc.subcore_barrier()
    pltpu.sync_copy(seg_acc.at[pl.ds(s_idx * RPT, RPT)],
                    seg_out.at[c_idx, pl.ds(s_idx * RPT, RPT)])
    plsc.subcore_barrier()

    hb = (hbuf0, hbuf1, hbuf2, hbuf3)
    pb = (pbuf0, pbuf1, pbuf2, pbuf3)
    gsem = (sg0, sg1, sg2, sg3)
    ssem = (ss0, ss1, ss2, ss3)
    for c in range(NCHUNK):
        href = hrefs[c]
        idx_c = jnp.full((16,), c, jnp.int32)

        def _p2_gather(b, slot):
            pltpu.async_copy(href.at[sidx2.at[b]], hb[slot], gsem[slot])
            pltpu.async_copy(
                p_out.at[pl.ds(ebase + b * BATCH, BATCH)], pb[slot],
                gsem[slot])

        _p2_gather(0, 0)
        _p2_gather(1, 1)

        def _gquad(g, _):
            for slot in range(4):
                b = 4 * g + slot
                pltpu.make_async_copy(
                    href.at[sidx2.at[b]], hb[slot], gsem[slot]).wait()
                pltpu.make_async_copy(
                    p_out.at[pl.ds(ebase + b * BATCH, BATCH)], pb[slot],
                    gsem[slot]).wait()

                def _quad(i, _):
                    for u in range(2):
                        e0 = 4 * i + 2 * u
                        e1 = e0 + 1
                        q0 = take(pb[slot][e0], idx_c)
                        q1 = take(pb[slot][e1], idx_c)
                        for j in range(CW // L):
                            s0 = pl.ds(j * L, L)
                            hb[slot][e0, s0] = hb[slot][e0, s0] * q0
                            hb[slot][e1, s0] = hb[slot][e1, s0] * q1
                    return 0

                lax.fori_loop(0, BATCH // 4, _quad, 0)
                pltpu.async_copy(
                    hb[slot], msg_acc.at[didx2.at[b]], ssem[slot], add=True)

                slot2 = (slot + 2) % 4

                @pl.when(b + 2 < eb_c)
                def _():
                    @pl.when(b >= 2)
                    def _():
                        pltpu.make_async_copy(
                            hb[slot2], msg_acc.at[didx2.at[b - 2]],
                            ssem[slot2]).wait()

                    _p2_gather(b + 2, slot2)
            return 0

        lax.fori_loop(0, eb_c // 4, _gquad, 0)
        for slot in range(4):
            pltpu.make_async_copy(
                hb[slot], msg_acc.at[didx2.at[eb_c - 4 + slot]],
                ssem[slot]).wait()
        plsc.subcore_barrier()
        pltpu.sync_copy(msg_acc.at[pl.ds(s_idx * RPT, RPT)],
                        msg_out.at[c_idx, c, pl.ds(s_idx * RPT, RPT)])
        if c < NCHUNK - 1:
            _zero_msg()
        plsc.subcore_barrier()


_sc_edges = pl.kernel(
    _sc_body,
    out_type=[
        jax.ShapeDtypeStruct((NC, NPAD, 16), jnp.float32),
        jax.ShapeDtypeStruct((NC, NCHUNK, NPAD, CW), jnp.float32),
        jax.ShapeDtypeStruct((EPAD, 16), jnp.float32),
    ],
    mesh=plsc.VectorSubcoreMesh(
        core_axis_name="c", subcore_axis_name="s",
        num_cores=NC, num_subcores=NS),
    scratch_types=[
        pltpu.VMEM((EBMAX, BATCH), jnp.int32),
        pltpu.VMEM((EBMAX, BATCH), jnp.int32),
        pltpu.VMEM((BATCH, 16), jnp.float32),
        pltpu.VMEM((BATCH, 16), jnp.float32),
        pltpu.VMEM((BATCH, 16), jnp.float32),
        pltpu.VMEM((BATCH, 16), jnp.float32),
        pltpu.VMEM((BATCH, 16), jnp.float32),
        pltpu.VMEM((BATCH, 16), jnp.float32),
        pltpu.VMEM((16,), jnp.float32),
        pltpu.VMEM((BATCH, CW), jnp.float32),
        pltpu.VMEM((BATCH, CW), jnp.float32),
        pltpu.VMEM((BATCH, CW), jnp.float32),
        pltpu.VMEM((BATCH, CW), jnp.float32),
        pltpu.VMEM((BATCH, 16), jnp.float32),
        pltpu.VMEM((BATCH, 16), jnp.float32),
        pltpu.VMEM((BATCH, 16), jnp.float32),
        pltpu.VMEM((BATCH, 16), jnp.float32),
        pltpu.VMEM((RPT // 20, 16), jnp.float32),
        pltpu.VMEM((RPT // 20, CW), jnp.float32),
        pltpu.VMEM_SHARED((NPAD, 16), jnp.float32),
        pltpu.VMEM_SHARED((NPAD, CW), jnp.float32),
    ] + [pltpu.SemaphoreType.DMA] * 8,
    compiler_params=pltpu.CompilerParams(use_tc_tiling_on_sc=False),
)

def _final_body(msg_ref, seg_ref, bias_ref, out_ref):
    seg = seg_ref[0] + seg_ref[1] + 1e-30
    cols = []
    for c in range(NCHUNK):
        m = msg_ref[0, c] + msg_ref[1, c]
        d = jnp.broadcast_to(seg[:, c:c + 1], (_ROWS, CW))
        cols.append(m / d)
    out_ref[...] = jnp.concatenate(cols, axis=1) + bias_ref[...]


def _finalize(msg, seg, bias2d):
    return pl.pallas_call(
        _final_body,
        grid=(_GRID,),
        in_specs=[
            pl.BlockSpec((NC, NCHUNK, _ROWS, CW), lambda i: (0, 0, i, 0)),
            pl.BlockSpec((NC, _ROWS, 16), lambda i: (0, i, 0)),
            pl.BlockSpec((1, HC), lambda i: (0, 0)),
        ],
        out_specs=pl.BlockSpec((_ROWS, HC), lambda i: (i, 0)),
        out_shape=jax.ShapeDtypeStruct((N, HC), jnp.float32),
    )(msg, seg, bias2d)


def kernel(x, edge_index, W, att_src, att_dst, bias):
    loop = jnp.arange(N, dtype=edge_index.dtype)
    ei = jnp.concatenate([edge_index, jnp.stack([loop, loop])], axis=1)
    ei = ei.astype(jnp.int32)
    src2 = jnp.pad(ei[0], (0, NBAT_ALLOC * BATCH - ETRUE)).reshape(-1, BATCH)
    dst2 = jnp.pad(ei[1], (0, NBAT_ALLOC * BATCH - ETRUE)).reshape(-1, BATCH)

    eye = jnp.eye(HEADS, dtype=jnp.float32)
    blk_s = att_src[:, :, None] * eye[:, None, :]
    blk_d = att_dst[:, :, None] * eye[:, None, :]
    A = jnp.concatenate([blk_s.reshape(HC, HEADS),
                         blk_d.reshape(HC, HEADS)], axis=1)

    outs = _project(x, W, A)
    hcs = outs[:NCHUNK]
    acat = outs[NCHUNK]
    macc16 = outs[NCHUNK + 1].reshape(16)

    seg, msg, _p = _sc_edges(acat, src2, dst2, macc16, *hcs)
    return _finalize(msg, seg, bias.reshape(1, HC))
